# Initial kernel scaffold; baseline (speedup 1.0000x reference)
import jax
import jax.numpy as jnp
from jax.experimental import pallas as pl


def kernel(x, edge_index, edge_attr, batch0, atom_emb, bond_emb, W1, b1, g1, be1, W2, b2, g2, be2, Wp1, bp1, Wp2, bp2):
    raise NotImplementedError("write your pallas kernel here")


# speedup vs baseline: 2.9652x; 2.9652x over previous
import functools

import jax
import jax.numpy as jnp
from jax import lax
from jax.experimental import pallas as pl
from jax.experimental.pallas import tpu as pltpu
from jax.experimental.pallas import tpu_sc as plsc

C = 128
L = 5
NUM_TASKS = 128
N = 10000
E = 320000
G = 512
VA = 119
VB = 8

_NWORK = 32
_EPW = E // _NWORK
_CHUNK = 80
_NCH = _EPW // _CHUNK
_RCH = N // _CHUNK


def _agg_body(h_hbm, e_hbm, src_hbm, dst_hbm, out_hbm,
              src_v, dst_v, ebuf, hbuf, aggr_sh, sem):
    cid = lax.axis_index("c")
    sid = lax.axis_index("s")
    wid = cid * 16 + sid

    zero = jnp.zeros((16,), jnp.float32)

    def zrow(r, carry):
        for b in range(8):
            ebuf[r, pl.ds(b * 16, 16)] = zero
        return carry

    lax.fori_loop(0, _CHUNK, zrow, 0)

    def zchunk(k, carry):
        rc = sid + k * 16

        @pl.when(rc < _RCH)
        def _():
            pltpu.sync_copy(ebuf, aggr_sh.at[pl.ds(rc * _CHUNK, _CHUNK)])

        return carry

    lax.fori_loop(0, (_RCH + 15) // 16, zchunk, 0)
    plsc.subcore_barrier()

    def chunk(j, carry):
        base = wid * _EPW + j * _CHUNK
        pltpu.sync_copy(src_hbm.at[wid, j], src_v)
        pltpu.sync_copy(dst_hbm.at[wid, j], dst_v)
        pltpu.sync_copy(e_hbm.at[pl.ds(base, _CHUNK)], ebuf)
        pltpu.async_copy(h_hbm.at[src_v], hbuf, sem).wait()

        def row(r, c2):
            for b in range(8):
                sl = pl.ds(b * 16, 16)
                ebuf[r, sl] = jnp.maximum(ebuf[r, sl] + hbuf[r, sl], 0.0)
            return c2

        lax.fori_loop(0, _CHUNK, row, 0)
        pltpu.sync_copy(ebuf, aggr_sh.at[dst_v], add=True)
        return carry

    lax.fori_loop(0, _NCH, chunk, 0)
    plsc.subcore_barrier()

    def wchunk(k, carry):
        rc = sid + k * 16

        @pl.when(rc < _RCH)
        def _():
            pltpu.sync_copy(aggr_sh.at[pl.ds(rc * _CHUNK, _CHUNK)],
                            out_hbm.at[cid, pl.ds(rc * _CHUNK, _CHUNK)])

        return carry

    lax.fori_loop(0, (_RCH + 15) // 16, wchunk, 0)


_aggregate = functools.partial(
    pl.kernel,
    out_type=jax.ShapeDtypeStruct((2, N, C), jnp.float32),
    mesh=plsc.VectorSubcoreMesh(core_axis_name="c", subcore_axis_name="s"),
    scratch_types=[
        pltpu.VMEM((_CHUNK,), jnp.int32),
        pltpu.VMEM((_CHUNK,), jnp.int32),
        pltpu.VMEM((_CHUNK, C), jnp.float32),
        pltpu.VMEM((_CHUNK, C), jnp.float32),
        pltpu.VMEM_SHARED((N, C), jnp.float32),
        pltpu.SemaphoreType.DMA,
    ],
)(_agg_body)


def _atom_body(x_ref, emb_ref, o_ref):
    acc = jnp.zeros(o_ref.shape, jnp.float32)
    for f in range(9):
        col = x_ref[:, f:f + 1]
        oh = (col == lax.broadcasted_iota(jnp.int32, (1, VA), 1))
        acc = acc + jnp.dot(oh.astype(jnp.float32), emb_ref[f],
                            preferred_element_type=jnp.float32, precision=lax.Precision.HIGHEST)
    o_ref[...] = acc


def _bond_body(a_ref, emb_ref, o_ref):
    acc = jnp.zeros(o_ref.shape, jnp.float32)
    for f in range(3):
        col = a_ref[:, f:f + 1]
        oh = (col == lax.broadcasted_iota(jnp.int32, (1, VB), 1))
        acc = acc + jnp.dot(oh.astype(jnp.float32), emb_ref[f],
                            preferred_element_type=jnp.float32, precision=lax.Precision.HIGHEST)
    o_ref[...] = acc


def _atom_encode(x, atom_emb):
    bn = 1000
    return pl.pallas_call(
        _atom_body,
        grid=(N // bn,),
        in_specs=[
            pl.BlockSpec((bn, 9), lambda i: (i, 0)),
            pl.BlockSpec((9, VA, C), lambda i: (0, 0, 0)),
        ],
        out_specs=pl.BlockSpec((bn, C), lambda i: (i, 0)),
        out_shape=jax.ShapeDtypeStruct((N, C), jnp.float32),
    )(x, atom_emb)


def _bond_encode(edge_attr, bond_emb):
    be = 2000
    return pl.pallas_call(
        _bond_body,
        grid=(E // be,),
        in_specs=[
            pl.BlockSpec((be, 3), lambda i: (i, 0)),
            pl.BlockSpec((3, VB, C), lambda i: (0, 0, 0)),
        ],
        out_specs=pl.BlockSpec((be, C), lambda i: (i, 0)),
        out_shape=jax.ShapeDtypeStruct((E, C), jnp.float32),
    )(edge_attr, bond_emb)


def _mlp_body(h_ref, p_ref, w1_ref, b1_ref, g1_ref, be1_ref,
              w2_ref, b2_ref, g2_ref, be2_ref, o_ref):
    z0 = h_ref[...] + p_ref[0] + p_ref[1]
    z1 = jnp.dot(z0, w1_ref[...], preferred_element_type=jnp.float32)
    z1 = z1 + b1_ref[...]
    mu = jnp.mean(z1, axis=0, keepdims=True)
    var = jnp.mean(jnp.square(z1 - mu), axis=0, keepdims=True)
    z1 = (z1 - mu) / jnp.sqrt(var + 1e-5) * g1_ref[...] + be1_ref[...]
    z1 = jnp.maximum(z1, 0.0)
    z2 = jnp.dot(z1, w2_ref[...], preferred_element_type=jnp.float32)
    z2 = z2 + b2_ref[...]
    mu2 = jnp.mean(z2, axis=0, keepdims=True)
    var2 = jnp.mean(jnp.square(z2 - mu2), axis=0, keepdims=True)
    z2 = (z2 - mu2) / jnp.sqrt(var2 + 1e-5) * g2_ref[...] + be2_ref[...]
    o_ref[...] = jnp.maximum(z2, 0.0)


def _mlp(h, p, w1, b1, g1, be1, w2, b2, g2, be2):
    return pl.pallas_call(
        _mlp_body,
        out_shape=jax.ShapeDtypeStruct((N, C), jnp.float32),
    )(h, p, w1, b1.reshape(1, -1), g1.reshape(1, -1), be1.reshape(1, -1),
      w2, b2.reshape(1, -1), g2.reshape(1, -1), be2.reshape(1, -1))


def _pool_body(h_ref, b_ref, wp1_ref, bp1_ref, wp2_ref, bp2_ref, o_ref):
    gid = lax.broadcasted_iota(jnp.int32, (G, 1), 0)
    oh = (b_ref[...] == gid).astype(jnp.float32)
    sums = jnp.dot(oh, h_ref[...], preferred_element_type=jnp.float32, precision=lax.Precision.HIGHEST)
    cnt = jnp.sum(oh, axis=1, keepdims=True)
    pooled = sums / jnp.maximum(cnt, 1.0)
    q = jnp.dot(pooled, wp1_ref[...], preferred_element_type=jnp.float32)
    q = q + bp1_ref[...]
    q = jnp.where(q > 0.0, q, jnp.exp(jnp.minimum(q, 0.0)) - 1.0)
    o_ref[...] = jnp.dot(q, wp2_ref[...],
                         preferred_element_type=jnp.float32) + bp2_ref[...]


def _pool(h, batch0, wp1, bp1, wp2, bp2):
    return pl.pallas_call(
        _pool_body,
        out_shape=jax.ShapeDtypeStruct((G, NUM_TASKS), jnp.float32),
    )(h, batch0, wp1, bp1.reshape(1, -1), wp2, bp2.reshape(1, -1))


def kernel(x, edge_index, edge_attr, batch0, atom_emb, bond_emb,
           W1, b1, g1, be1, W2, b2, g2, be2, Wp1, bp1, Wp2, bp2):
    x = x.astype(jnp.int32)
    edge_attr = edge_attr.astype(jnp.int32)
    src = edge_index[0].astype(jnp.int32).reshape(_NWORK, _NCH, _CHUNK)
    dst = edge_index[1].astype(jnp.int32).reshape(_NWORK, _NCH, _CHUNK)
    b0 = batch0.astype(jnp.int32).reshape(1, N)

    h = _atom_encode(x, atom_emb)
    e = _bond_encode(edge_attr, bond_emb)

    for l in range(L):
        p = _aggregate(h, e, src, dst)
        h = _mlp(h, p, W1[l], b1[l], g1[l], be1[l],
                 W2[l], b2[l], g2[l], be2[l])

    out = _pool(h, b0, Wp1, bp1, Wp2, bp2)
    return out.reshape((-1, NUM_TASKS))


# speedup vs baseline: 5.3340x; 1.7988x over previous
import functools

import jax
import jax.numpy as jnp
from jax import lax
from jax.experimental import pallas as pl
from jax.experimental.pallas import tpu as pltpu
from jax.experimental.pallas import tpu_sc as plsc

C = 128
L = 5
NUM_TASKS = 128
N = 10000
E = 320000
G = 512
VA = 119
VB = 8

_NWORK = 32
_EPW = E // _NWORK
_CHUNK = 40
_NCH = _EPW // _CHUNK
_NHALF = 2
_HCH = _NCH // _NHALF
_RCH = N // _CHUNK


def _agg_body(h_hbm, e_hbm, sd_hbm, z_hbm, out_hbm,
              sd_v, src0, dst0, src1, dst1,
              ebuf0, ebuf1, hbuf0, hbuf1, aggr_sh,
              se0, se1, sg0, sg1):
    cid = lax.axis_index("c")
    sid = lax.axis_index("s")
    wid = cid * 16 + sid

    @pl.when(sid < 15)
    def _():
        pltpu.sync_copy(z_hbm.at[pl.ds(0, 624)],
                        aggr_sh.at[pl.ds(sid * 624, 624)])

    @pl.when(sid == 15)
    def _():
        pltpu.sync_copy(z_hbm, aggr_sh.at[pl.ds(9360, 640)])

    plsc.subcore_barrier()

    def issue(j, half, eb, hb, se, sg, src_i, dst_i):
        for off in (0, 16, 24):
            p = sd_v[j, pl.ds(off, 16)]
            src_i[pl.ds(off, 16)] = p & 0xFFFF
            dst_i[pl.ds(off, 16)] = p >> 16
        base = wid * _EPW + half * (_HCH * _CHUNK) + j * _CHUNK
        pltpu.async_copy(e_hbm.at[pl.ds(base, _CHUNK)], eb, se)
        pltpu.async_copy(h_hbm.at[src_i], hb, sg)

    def process(j, eb, hb, se, sg, dst_i):
        pltpu.make_async_copy(e_hbm.at[pl.ds(0, _CHUNK)], eb, se).wait()
        pltpu.make_async_copy(h_hbm.at[src0], hb, sg).wait()

        @plsc.parallel_loop(0, _CHUNK, step=1, unroll=4)
        def row(r):
            for b in range(8):
                sl = pl.ds(b * 16, 16)
                eb[r, sl] = jnp.maximum(eb[r, sl] + hb[r, sl], 0.0)

        pltpu.sync_copy(eb, aggr_sh.at[dst_i], add=True)

    for half in range(_NHALF):
        pltpu.sync_copy(sd_hbm.at[wid, half], sd_v)
        issue(0, half, ebuf0, hbuf0, se0, sg0, src0, dst0)
        issue(1, half, ebuf1, hbuf1, se1, sg1, src1, dst1)

        def pair(i, carry):
            j0 = 2 * i
            process(j0, ebuf0, hbuf0, se0, sg0, dst0)
            issue(j0 + 2, half, ebuf0, hbuf0, se0, sg0, src0, dst0)
            process(j0 + 1, ebuf1, hbuf1, se1, sg1, dst1)

            @pl.when(i < _HCH // 2 - 1)
            def _():
                issue(j0 + 3, half, ebuf1, hbuf1, se1, sg1, src1, dst1)

            return carry

        lax.fori_loop(0, _HCH // 2, pair, 0)
        process(_HCH - 1, ebuf0, hbuf0, se0, sg0, dst0)

    plsc.subcore_barrier()

    @pl.when(sid < 15)
    def _():
        pltpu.sync_copy(aggr_sh.at[pl.ds(sid * 624, 624)],
                        out_hbm.at[cid, pl.ds(sid * 624, 624)])

    @pl.when(sid == 15)
    def _():
        pltpu.sync_copy(aggr_sh.at[pl.ds(9360, 640)],
                        out_hbm.at[cid, pl.ds(9360, 640)])


_aggregate = functools.partial(
    pl.kernel,
    out_type=jax.ShapeDtypeStruct((2, N, C), jnp.float32),
    mesh=plsc.VectorSubcoreMesh(core_axis_name="c", subcore_axis_name="s"),
    scratch_types=[
        pltpu.VMEM((_HCH, _CHUNK), jnp.int32),
        pltpu.VMEM((_CHUNK,), jnp.int32),
        pltpu.VMEM((_CHUNK,), jnp.int32),
        pltpu.VMEM((_CHUNK,), jnp.int32),
        pltpu.VMEM((_CHUNK,), jnp.int32),
        pltpu.VMEM((_CHUNK, C), jnp.float32),
        pltpu.VMEM((_CHUNK, C), jnp.float32),
        pltpu.VMEM((_CHUNK, C), jnp.float32),
        pltpu.VMEM((_CHUNK, C), jnp.float32),
        pltpu.VMEM_SHARED((N, C), jnp.float32),
        pltpu.SemaphoreType.DMA,
        pltpu.SemaphoreType.DMA,
        pltpu.SemaphoreType.DMA,
        pltpu.SemaphoreType.DMA,
    ],
)(_agg_body)


def _atom_body(x_ref, emb_ref, o_ref):
    acc = jnp.zeros(o_ref.shape, jnp.float32)
    for f in range(9):
        col = x_ref[:, f:f + 1]
        oh = (col == lax.broadcasted_iota(jnp.int32, (1, VA), 1))
        acc = acc + jnp.dot(oh.astype(jnp.float32), emb_ref[f],
                            preferred_element_type=jnp.float32, precision=lax.Precision.HIGHEST)
    o_ref[...] = acc


def _bond_body(a_ref, emb_ref, o_ref):
    acc = jnp.zeros(o_ref.shape, jnp.float32)
    for f in range(3):
        col = a_ref[:, f:f + 1]
        oh = (col == lax.broadcasted_iota(jnp.int32, (1, VB), 1))
        acc = acc + jnp.dot(oh.astype(jnp.float32), emb_ref[f],
                            preferred_element_type=jnp.float32, precision=lax.Precision.HIGHEST)
    o_ref[...] = acc


def _atom_encode(x, atom_emb):
    bn = 1000
    return pl.pallas_call(
        _atom_body,
        grid=(N // bn,),
        in_specs=[
            pl.BlockSpec((bn, 9), lambda i: (i, 0)),
            pl.BlockSpec((9, VA, C), lambda i: (0, 0, 0)),
        ],
        out_specs=pl.BlockSpec((bn, C), lambda i: (i, 0)),
        out_shape=jax.ShapeDtypeStruct((N, C), jnp.float32),
    )(x, atom_emb)


def _bond_encode(edge_attr, bond_emb):
    be = 2000
    return pl.pallas_call(
        _bond_body,
        grid=(E // be,),
        in_specs=[
            pl.BlockSpec((be, 3), lambda i: (i, 0)),
            pl.BlockSpec((3, VB, C), lambda i: (0, 0, 0)),
        ],
        out_specs=pl.BlockSpec((be, C), lambda i: (i, 0)),
        out_shape=jax.ShapeDtypeStruct((E, C), jnp.float32),
    )(edge_attr, bond_emb)


def _mlp_body(h_ref, p_ref, w1_ref, b1_ref, g1_ref, be1_ref,
              w2_ref, b2_ref, g2_ref, be2_ref, o_ref):
    z0 = h_ref[...] + p_ref[0] + p_ref[1]
    z1 = jnp.dot(z0, w1_ref[...], preferred_element_type=jnp.float32)
    z1 = z1 + b1_ref[...]
    mu = jnp.mean(z1, axis=0, keepdims=True)
    var = jnp.mean(jnp.square(z1 - mu), axis=0, keepdims=True)
    z1 = (z1 - mu) / jnp.sqrt(var + 1e-5) * g1_ref[...] + be1_ref[...]
    z1 = jnp.maximum(z1, 0.0)
    z2 = jnp.dot(z1, w2_ref[...], preferred_element_type=jnp.float32)
    z2 = z2 + b2_ref[...]
    mu2 = jnp.mean(z2, axis=0, keepdims=True)
    var2 = jnp.mean(jnp.square(z2 - mu2), axis=0, keepdims=True)
    z2 = (z2 - mu2) / jnp.sqrt(var2 + 1e-5) * g2_ref[...] + be2_ref[...]
    o_ref[...] = jnp.maximum(z2, 0.0)


def _mlp(h, p, w1, b1, g1, be1, w2, b2, g2, be2):
    return pl.pallas_call(
        _mlp_body,
        out_shape=jax.ShapeDtypeStruct((N, C), jnp.float32),
    )(h, p, w1, b1.reshape(1, -1), g1.reshape(1, -1), be1.reshape(1, -1),
      w2, b2.reshape(1, -1), g2.reshape(1, -1), be2.reshape(1, -1))


def _pool_body(h_ref, b_ref, wp1_ref, bp1_ref, wp2_ref, bp2_ref, o_ref):
    gid = lax.broadcasted_iota(jnp.int32, (G, 1), 0)
    oh = (b_ref[...] == gid).astype(jnp.float32)
    sums = jnp.dot(oh, h_ref[...], preferred_element_type=jnp.float32, precision=lax.Precision.HIGHEST)
    cnt = jnp.sum(oh, axis=1, keepdims=True)
    pooled = sums / jnp.maximum(cnt, 1.0)
    q = jnp.dot(pooled, wp1_ref[...], preferred_element_type=jnp.float32)
    q = q + bp1_ref[...]
    q = jnp.where(q > 0.0, q, jnp.exp(jnp.minimum(q, 0.0)) - 1.0)
    o_ref[...] = jnp.dot(q, wp2_ref[...],
                         preferred_element_type=jnp.float32) + bp2_ref[...]


def _pool(h, batch0, wp1, bp1, wp2, bp2):
    return pl.pallas_call(
        _pool_body,
        out_shape=jax.ShapeDtypeStruct((G, NUM_TASKS), jnp.float32),
    )(h, batch0, wp1, bp1.reshape(1, -1), wp2, bp2.reshape(1, -1))


def kernel(x, edge_index, edge_attr, batch0, atom_emb, bond_emb,
           W1, b1, g1, be1, W2, b2, g2, be2, Wp1, bp1, Wp2, bp2):
    x = x.astype(jnp.int32)
    edge_attr = edge_attr.astype(jnp.int32)
    src = edge_index[0].astype(jnp.int32)
    dst = edge_index[1].astype(jnp.int32)
    sd = (src | (dst << 16)).reshape(_NWORK, _NHALF, _HCH, _CHUNK)
    zrows = jnp.zeros((640, C), jnp.float32)
    b0 = batch0.astype(jnp.int32).reshape(1, N)

    h = _atom_encode(x, atom_emb)
    e = _bond_encode(edge_attr, bond_emb)

    for l in range(L):
        p = _aggregate(h, e, sd, zrows)
        h = _mlp(h, p, W1[l], b1[l], g1[l], be1[l],
                 W2[l], b2[l], g2[l], be2[l])

    out = _pool(h, b0, Wp1, bp1, Wp2, bp2)
    return out.reshape((-1, NUM_TASKS))


# speedup vs baseline: 5.5767x; 1.0455x over previous
import functools

import jax
import jax.numpy as jnp
from jax import lax
from jax.experimental import pallas as pl
from jax.experimental.pallas import tpu as pltpu
from jax.experimental.pallas import tpu_sc as plsc

C = 128
L = 5
NUM_TASKS = 128
N = 10000
E = 320000
G = 512
VA = 119
VB = 8

_NWORK = 32
_EPW = E // _NWORK
_CHUNK = 80
_NCH = _EPW // _CHUNK
_NSLAB = 5
_SCH = _NCH // _NSLAB


def _agg_body(h_hbm, e_hbm, sd_hbm, z_hbm, out_hbm,
              sd_v, src0, dst0, src1, dst1,
              ebuf0, ebuf1, hbuf0, hbuf1, aggr_sh,
              se0, se1, sg0, sg1):
    cid = lax.axis_index("c")
    sid = lax.axis_index("s")
    wid = cid * 16 + sid

    @pl.when(sid < 15)
    def _():
        pltpu.sync_copy(z_hbm.at[pl.ds(0, 624)],
                        aggr_sh.at[pl.ds(sid * 624, 624)])

    @pl.when(sid == 15)
    def _():
        pltpu.sync_copy(z_hbm, aggr_sh.at[pl.ds(9360, 640)])

    plsc.subcore_barrier()

    def issue(j, slab, eb, hb, se, sg, src_i, dst_i):
        for off in (0, 16, 32, 48, 64):
            p = sd_v[j, pl.ds(off, 16)]
            src_i[pl.ds(off, 16)] = p & 0xFFFF
            dst_i[pl.ds(off, 16)] = p >> 16
        base = wid * _EPW + slab * (_SCH * _CHUNK) + j * _CHUNK
        pltpu.async_copy(e_hbm.at[pl.ds(base, _CHUNK)], eb, se)
        pltpu.async_copy(h_hbm.at[src_i], hb, sg)

    def process(j, eb, hb, se, sg, dst_i):
        pltpu.make_async_copy(e_hbm.at[pl.ds(0, _CHUNK)], eb, se).wait()
        pltpu.make_async_copy(h_hbm.at[src0], hb, sg).wait()

        @plsc.parallel_loop(0, _CHUNK, step=1, unroll=4)
        def row(r):
            for b in range(8):
                sl = pl.ds(b * 16, 16)
                eb[r, sl] = jnp.maximum(eb[r, sl] + hb[r, sl], 0.0)

        pltpu.sync_copy(eb, aggr_sh.at[dst_i], add=True)

    for slab in range(_NSLAB):
        pltpu.sync_copy(sd_hbm.at[wid, slab], sd_v)
        issue(0, slab, ebuf0, hbuf0, se0, sg0, src0, dst0)
        issue(1, slab, ebuf1, hbuf1, se1, sg1, src1, dst1)

        def pair(i, carry):
            j0 = 2 * i
            process(j0, ebuf0, hbuf0, se0, sg0, dst0)
            issue(j0 + 2, slab, ebuf0, hbuf0, se0, sg0, src0, dst0)
            process(j0 + 1, ebuf1, hbuf1, se1, sg1, dst1)

            @pl.when(i < _SCH // 2 - 1)
            def _():
                issue(j0 + 3, slab, ebuf1, hbuf1, se1, sg1, src1, dst1)

            return carry

        lax.fori_loop(0, _SCH // 2, pair, 0)
        process(_SCH - 1, ebuf0, hbuf0, se0, sg0, dst0)

    plsc.subcore_barrier()

    @pl.when(sid < 15)
    def _():
        pltpu.sync_copy(aggr_sh.at[pl.ds(sid * 624, 624)],
                        out_hbm.at[cid, pl.ds(sid * 624, 624)])

    @pl.when(sid == 15)
    def _():
        pltpu.sync_copy(aggr_sh.at[pl.ds(9360, 640)],
                        out_hbm.at[cid, pl.ds(9360, 640)])


_aggregate = functools.partial(
    pl.kernel,
    out_type=jax.ShapeDtypeStruct((2, N, C), jnp.float32),
    mesh=plsc.VectorSubcoreMesh(core_axis_name="c", subcore_axis_name="s"),
    scratch_types=[
        pltpu.VMEM((_SCH, _CHUNK), jnp.int32),
        pltpu.VMEM((_CHUNK,), jnp.int32),
        pltpu.VMEM((_CHUNK,), jnp.int32),
        pltpu.VMEM((_CHUNK,), jnp.int32),
        pltpu.VMEM((_CHUNK,), jnp.int32),
        pltpu.VMEM((_CHUNK, C), jnp.float32),
        pltpu.VMEM((_CHUNK, C), jnp.float32),
        pltpu.VMEM((_CHUNK, C), jnp.float32),
        pltpu.VMEM((_CHUNK, C), jnp.float32),
        pltpu.VMEM_SHARED((N, C), jnp.float32),
        pltpu.SemaphoreType.DMA,
        pltpu.SemaphoreType.DMA,
        pltpu.SemaphoreType.DMA,
        pltpu.SemaphoreType.DMA,
    ],
)(_agg_body)


def _atom_body(x_ref, emb_ref, o_ref):
    acc = jnp.zeros(o_ref.shape, jnp.float32)
    for f in range(9):
        col = x_ref[:, f:f + 1]
        oh = (col == lax.broadcasted_iota(jnp.int32, (1, VA), 1))
        acc = acc + jnp.dot(oh.astype(jnp.float32), emb_ref[f],
                            preferred_element_type=jnp.float32, precision=lax.Precision.HIGHEST)
    o_ref[...] = acc


def _bond_body(a_ref, emb_ref, o_ref):
    acc = jnp.zeros(o_ref.shape, jnp.float32)
    for f in range(3):
        col = a_ref[:, f:f + 1]
        oh = (col == lax.broadcasted_iota(jnp.int32, (1, VB), 1))
        acc = acc + jnp.dot(oh.astype(jnp.float32), emb_ref[f],
                            preferred_element_type=jnp.float32, precision=lax.Precision.HIGHEST)
    o_ref[...] = acc


def _atom_encode(x, atom_emb):
    bn = 1000
    return pl.pallas_call(
        _atom_body,
        grid=(N // bn,),
        in_specs=[
            pl.BlockSpec((bn, 9), lambda i: (i, 0)),
            pl.BlockSpec((9, VA, C), lambda i: (0, 0, 0)),
        ],
        out_specs=pl.BlockSpec((bn, C), lambda i: (i, 0)),
        out_shape=jax.ShapeDtypeStruct((N, C), jnp.float32),
    )(x, atom_emb)


def _bond_encode(edge_attr, bond_emb):
    be = 2000
    return pl.pallas_call(
        _bond_body,
        grid=(E // be,),
        in_specs=[
            pl.BlockSpec((be, 3), lambda i: (i, 0)),
            pl.BlockSpec((3, VB, C), lambda i: (0, 0, 0)),
        ],
        out_specs=pl.BlockSpec((be, C), lambda i: (i, 0)),
        out_shape=jax.ShapeDtypeStruct((E, C), jnp.float32),
    )(edge_attr, bond_emb)


def _mlp_body(h_ref, p_ref, w1_ref, b1_ref, g1_ref, be1_ref,
              w2_ref, b2_ref, g2_ref, be2_ref, o_ref):
    z0 = h_ref[...] + p_ref[0] + p_ref[1]
    z1 = jnp.dot(z0, w1_ref[...], preferred_element_type=jnp.float32)
    z1 = z1 + b1_ref[...]
    mu = jnp.mean(z1, axis=0, keepdims=True)
    var = jnp.mean(jnp.square(z1 - mu), axis=0, keepdims=True)
    z1 = (z1 - mu) / jnp.sqrt(var + 1e-5) * g1_ref[...] + be1_ref[...]
    z1 = jnp.maximum(z1, 0.0)
    z2 = jnp.dot(z1, w2_ref[...], preferred_element_type=jnp.float32)
    z2 = z2 + b2_ref[...]
    mu2 = jnp.mean(z2, axis=0, keepdims=True)
    var2 = jnp.mean(jnp.square(z2 - mu2), axis=0, keepdims=True)
    z2 = (z2 - mu2) / jnp.sqrt(var2 + 1e-5) * g2_ref[...] + be2_ref[...]
    o_ref[...] = jnp.maximum(z2, 0.0)


def _mlp(h, p, w1, b1, g1, be1, w2, b2, g2, be2):
    return pl.pallas_call(
        _mlp_body,
        out_shape=jax.ShapeDtypeStruct((N, C), jnp.float32),
    )(h, p, w1, b1.reshape(1, -1), g1.reshape(1, -1), be1.reshape(1, -1),
      w2, b2.reshape(1, -1), g2.reshape(1, -1), be2.reshape(1, -1))


def _pool_body(h_ref, b_ref, wp1_ref, bp1_ref, wp2_ref, bp2_ref, o_ref):
    gid = lax.broadcasted_iota(jnp.int32, (G, 1), 0)
    oh = (b_ref[...] == gid).astype(jnp.float32)
    sums = jnp.dot(oh, h_ref[...], preferred_element_type=jnp.float32, precision=lax.Precision.HIGHEST)
    cnt = jnp.sum(oh, axis=1, keepdims=True)
    pooled = sums / jnp.maximum(cnt, 1.0)
    q = jnp.dot(pooled, wp1_ref[...], preferred_element_type=jnp.float32)
    q = q + bp1_ref[...]
    q = jnp.where(q > 0.0, q, jnp.exp(jnp.minimum(q, 0.0)) - 1.0)
    o_ref[...] = jnp.dot(q, wp2_ref[...],
                         preferred_element_type=jnp.float32) + bp2_ref[...]


def _pool(h, batch0, wp1, bp1, wp2, bp2):
    return pl.pallas_call(
        _pool_body,
        out_shape=jax.ShapeDtypeStruct((G, NUM_TASKS), jnp.float32),
    )(h, batch0, wp1, bp1.reshape(1, -1), wp2, bp2.reshape(1, -1))


def kernel(x, edge_index, edge_attr, batch0, atom_emb, bond_emb,
           W1, b1, g1, be1, W2, b2, g2, be2, Wp1, bp1, Wp2, bp2):
    x = x.astype(jnp.int32)
    edge_attr = edge_attr.astype(jnp.int32)
    src = edge_index[0].astype(jnp.int32)
    dst = edge_index[1].astype(jnp.int32)
    sd = (src | (dst << 16)).reshape(_NWORK, _NSLAB, _SCH, _CHUNK)
    zrows = jnp.zeros((640, C), jnp.float32)
    b0 = batch0.astype(jnp.int32).reshape(1, N)

    h = _atom_encode(x, atom_emb)
    e = _bond_encode(edge_attr, bond_emb)

    for l in range(L):
        p = _aggregate(h, e, sd, zrows)
        h = _mlp(h, p, W1[l], b1[l], g1[l], be1[l],
                 W2[l], b2[l], g2[l], be2[l])

    out = _pool(h, b0, Wp1, bp1, Wp2, bp2)
    return out.reshape((-1, NUM_TASKS))
